# baseline (speedup 1.0000x reference)
import functools

import jax
import jax.numpy as jnp
from jax import lax
from jax.experimental import pallas as pl
from jax.experimental.pallas import tpu as pltpu
from jax.experimental.pallas import tpu_sc as plsc

B, L, V, D = 16384, 20, 1000000, 64
H1, H2 = 128, 64
OFF_V = 503808
NBLK = 244

NC, NS = 2, 16
NW = NC * NS
BAGS_W = B // NW
IG = 128
JB = 256
PAIRS_W = BAGS_W // 2
FB = 1024
BR2 = 4096

_mesh = plsc.VectorSubcoreMesh(core_axis_name="c", subcore_axis_name="s")


def _fmt_body(qi, ci, qo, co):
    qo[...] = qi[...].reshape(L, FB // IG, IG)
    co[...] = ci[...].reshape(L, FB // IG, IG)


_fmt_call = pl.pallas_call(
    _fmt_body,
    grid=(B // FB,),
    in_specs=[
        pl.BlockSpec((L, FB), lambda i: (0, i)),
        pl.BlockSpec((L, FB), lambda i: (0, i)),
    ],
    out_specs=[
        pl.BlockSpec((L, FB // IG, IG), lambda i: (0, i, 0)),
        pl.BlockSpec((L, FB // IG, IG), lambda i: (0, i, 0)),
    ],
    out_shape=[
        jax.ShapeDtypeStruct((L, B // IG, IG), jnp.int32),
        jax.ShapeDtypeStruct((L, B // IG, IG), jnp.int32),
    ],
)


def _pack_body(qa, qb, ca, cb, qo, co):
    qo[...] = jnp.concatenate([qa[...].T, qb[...].T], axis=1)
    co[...] = jnp.concatenate([ca[...].T, cb[...].T], axis=1)


_pack_call = pl.pallas_call(
    _pack_body,
    grid=(OFF_V // BR2,),
    in_specs=[
        pl.BlockSpec((D, BR2), lambda i: (0, i)),
        pl.BlockSpec((D, BR2), lambda i: (0, jnp.minimum(i + OFF_V // BR2, NBLK))),
        pl.BlockSpec((D, BR2), lambda i: (0, i)),
        pl.BlockSpec((D, BR2), lambda i: (0, jnp.minimum(i + OFF_V // BR2, NBLK))),
    ],
    out_specs=[
        pl.BlockSpec((BR2, 2 * D), lambda i: (i, 0)),
        pl.BlockSpec((BR2, 2 * D), lambda i: (i, 0)),
    ],
    out_shape=[
        jax.ShapeDtypeStruct((OFF_V, 2 * D), jnp.float32),
        jax.ShapeDtypeStruct((OFF_V, 2 * D), jnp.float32),
    ],
)


@functools.partial(
    pl.kernel,
    out_type=(
        jax.ShapeDtypeStruct((B // 2, 2 * D), jnp.float32),
        jax.ShapeDtypeStruct((B // 2, 2 * D), jnp.float32),
    ),
    mesh=_mesh,
    scratch_types=[
        pltpu.VMEM((8, IG), jnp.int32),
        pltpu.VMEM((8, IG), jnp.int32),
        pltpu.VMEM((JB,), jnp.int32),
        pltpu.VMEM((JB,), jnp.int32),
        pltpu.VMEM((JB, 2 * D), jnp.float32),
        pltpu.VMEM((JB, 2 * D), jnp.float32),
        pltpu.VMEM((PAIRS_W, 2 * D), jnp.float32),
        pltpu.SemaphoreType.DMA,
        pltpu.SemaphoreType.DMA,
        pltpu.SemaphoreType.DMA,
        pltpu.SemaphoreType.DMA,
    ],
    compiler_params=pltpu.CompilerParams(
        use_tc_tiling_on_sc=True, needs_layout_passes=False),
)
def _pool_kernel(qidx_hbm, cidx_hbm, tq_hbm, tc_hbm, qout_hbm, cout_hbm,
                 ist0, ist1, sh0, sh1, rows0, rows1, acc_v,
                 si0, si1, sr0, sr1):
    wid = lax.axis_index("s") * NC + lax.axis_index("c")
    prow = pl.multiple_of((wid & ~jnp.int32(1)) * 4, 8)
    srb = (wid & 1) * 4
    ists = (ist0, ist1)
    shs = (sh0, sh1)
    rows = (rows0, rows1)
    sis = (si0, si1)
    srs = (sr0, sr1)
    iota16 = lax.iota(jnp.int32, 16)
    last = jnp.int32(L - 1)

    def plane_copy(idx_hbm, l, pb):
        return pltpu.make_async_copy(
            idx_hbm.at[jnp.minimum(l, last), pl.ds(prow, 8), :], ists[pb], sis[pb])

    def gather_copies(tab_hbm, jb):
        return [
            pltpu.make_async_copy(
                tab_hbm.at[shs[jb].at[pl.ds(g * IG, IG)]],
                rows[jb].at[pl.ds(g * IG, IG), :],
                srs[jb],
            )
            for g in range(JB // IG)
        ]

    def shift(h, pb, jb):
        for k in range(16):
            v = ists[pb][srb + h * 2 + k // 8, pl.ds((k % 8) * 16, 16)]
            shs[jb][pl.ds(k * 16, 16)] = jnp.where(v >= OFF_V, v - OFF_V, v)

    def fire(tab_hbm, jb):
        for cp in gather_copies(tab_hbm, jb):
            cp.start()

    def drain(tab_hbm, jb):
        for cp in gather_copies(tab_hbm, jb):
            cp.wait()

    def accumulate(h, pb, jb):
        rv = rows[jb]
        pv = ists[pb]

        def abody(i2, _):
            for u in range(2):
                ri = i2 * 2 + u
                srow = srb + h * 2 + (ri >> 7)
                lane = ri & 127
                raw = plsc.load_gather(
                    pv, [jnp.full((16,), srow, jnp.int32),
                         jnp.full((16,), lane, jnp.int32)])
                hi_m = raw >= OFF_V
                arow = h * (JB // 2) + i2
                for d in range(4):
                    lo = rv[ri, pl.ds(d * 16, 16)]
                    hi = rv[ri, pl.ds(D + d * 16, 16)]
                    val = jnp.where(hi_m, hi, lo)
                    asl = pl.ds(u * D + d * 16, 16)
                    acc_v[arow, asl] = acc_v[arow, asl] + val
            return 0

        lax.fori_loop(0, JB // 2, abody, 0, unroll=8)

    def zero_acc():
        z = jnp.zeros((16,), jnp.float32)

        def zbody(r, _):
            for c8 in range(8):
                acc_v[r, pl.ds(c8 * 16, 16)] = z
            return 0

        lax.fori_loop(0, PAIRS_W, zbody, 0, unroll=False)

    for idx_hbm, tab_hbm, out_hbm in (
        (qidx_hbm, tq_hbm, qout_hbm),
        (cidx_hbm, tc_hbm, cout_hbm),
    ):
        zero_acc()
        cp = plane_copy(idx_hbm, jnp.int32(0), 0)
        cp.start()
        cp.wait()
        plane_copy(idx_hbm, jnp.int32(1), 1).start()
        shift(0, 0, 0)
        fire(tab_hbm, 0)

        def body(l2, _, idx_hbm=idx_hbm, tab_hbm=tab_hbm):
            sl = l2 * 2
            shift(1, 0, 1)
            fire(tab_hbm, 1)
            drain(tab_hbm, 0)
            accumulate(0, 0, 0)
            plane_copy(idx_hbm, sl + 1, 1).wait()
            shift(0, 1, 0)
            fire(tab_hbm, 0)
            drain(tab_hbm, 1)
            accumulate(1, 0, 1)
            plane_copy(idx_hbm, sl + 2, 0).start()
            shift(1, 1, 1)
            fire(tab_hbm, 1)
            drain(tab_hbm, 0)
            accumulate(0, 1, 0)
            plane_copy(idx_hbm, sl + 2, 0).wait()
            shift(0, 0, 0)
            fire(tab_hbm, 0)
            drain(tab_hbm, 1)
            accumulate(1, 1, 1)
            plane_copy(idx_hbm, sl + 3, 1).start()
            return 0

        lax.fori_loop(0, L // 2, body, 0, unroll=False)
        drain(tab_hbm, 0)
        plane_copy(idx_hbm, last, 1).wait()
        pltpu.sync_copy(
            acc_v,
            out_hbm.at[pl.ds(pl.multiple_of(wid * PAIRS_W, 8), PAIRS_W), :])


BMP = 2048


def _mlp_body(qx, cx, qw1, qb1, qw2, qb2, cw1, cb1, cw2, cb2, qo, co):
    def tower(xp, w1, b1, w2, b2, out_ref):
        xe = xp[:, :D]
        xo = xp[:, D:]
        x = jnp.concatenate([xe, xo], axis=0)
        h = jnp.maximum(
            jnp.dot(x, w1[...], preferred_element_type=jnp.float32) + b1[...], 0.0)
        y = jnp.maximum(
            jnp.dot(h, w2[...], preferred_element_type=jnp.float32) + b2[...], 0.0)
        ye = y[:BMP]
        yo = y[BMP:]
        out_ref[...] = jnp.stack([ye, yo], axis=1).reshape(2 * BMP, H2)

    tower(qx[...], qw1, qb1, qw2, qb2, qo)
    tower(cx[...], cw1, cb1, cw2, cb2, co)


_mlp_call = pl.pallas_call(
    _mlp_body,
    grid=(B // (2 * BMP),),
    in_specs=[
        pl.BlockSpec((BMP, 2 * D), lambda i: (i, 0)),
        pl.BlockSpec((BMP, 2 * D), lambda i: (i, 0)),
        pl.BlockSpec((D, H1), lambda i: (0, 0)),
        pl.BlockSpec((1, H1), lambda i: (0, 0)),
        pl.BlockSpec((H1, H2), lambda i: (0, 0)),
        pl.BlockSpec((1, H2), lambda i: (0, 0)),
        pl.BlockSpec((D, H1), lambda i: (0, 0)),
        pl.BlockSpec((1, H1), lambda i: (0, 0)),
        pl.BlockSpec((H1, H2), lambda i: (0, 0)),
        pl.BlockSpec((1, H2), lambda i: (0, 0)),
    ],
    out_specs=[
        pl.BlockSpec((2 * BMP, H2), lambda i: (i, 0)),
        pl.BlockSpec((2 * BMP, H2), lambda i: (i, 0)),
    ],
    out_shape=[
        jax.ShapeDtypeStruct((B, H2), jnp.float32),
        jax.ShapeDtypeStruct((B, H2), jnp.float32),
    ],
)


def kernel(query_indices, candidate_indices, table_query, table_candidate,
           q_w1, q_b1, q_w2, q_b2, c_w1, c_b1, c_w2, c_b2):
    qidx, cidx = _fmt_call(query_indices.T.astype(jnp.int32),
                           candidate_indices.T.astype(jnp.int32))
    tqT = table_query.T
    tcT = table_candidate.T
    tq2, tc2 = _pack_call(tqT, tqT, tcT, tcT)
    q_pooled, c_pooled = _pool_kernel(qidx, cidx, tq2, tc2)
    qe, ce = _mlp_call(
        q_pooled, c_pooled,
        q_w1, q_b1.reshape(1, H1), q_w2, q_b2.reshape(1, H2),
        c_w1, c_b1.reshape(1, H1), c_w2, c_b2.reshape(1, H2),
    )
    return (qe, ce)


# speedup vs baseline: 1.0067x; 1.0067x over previous
import functools

import jax
import jax.numpy as jnp
from jax import lax
from jax.experimental import pallas as pl
from jax.experimental.pallas import tpu as pltpu
from jax.experimental.pallas import tpu_sc as plsc

B, L, V, D = 16384, 20, 1000000, 64
H1, H2 = 128, 64
OFF_V = 503808
NBLK = 244

NC, NS = 2, 16
NW = NC * NS
BAGS_W = B // NW
IG = 128
JB = 256
PAIRS_W = BAGS_W // 2
FB = 1024
BR2 = 4096

_mesh = plsc.VectorSubcoreMesh(core_axis_name="c", subcore_axis_name="s")


def _fmt_body(qi, ci, qo, co):
    qo[...] = qi[...].reshape(L, FB // IG, IG)
    co[...] = ci[...].reshape(L, FB // IG, IG)


_fmt_call = pl.pallas_call(
    _fmt_body,
    grid=(B // FB,),
    in_specs=[
        pl.BlockSpec((L, FB), lambda i: (0, i)),
        pl.BlockSpec((L, FB), lambda i: (0, i)),
    ],
    out_specs=[
        pl.BlockSpec((L, FB // IG, IG), lambda i: (0, i, 0)),
        pl.BlockSpec((L, FB // IG, IG), lambda i: (0, i, 0)),
    ],
    out_shape=[
        jax.ShapeDtypeStruct((L, B // IG, IG), jnp.int32),
        jax.ShapeDtypeStruct((L, B // IG, IG), jnp.int32),
    ],
)


def _pack_body(qa, qb, ca, cb, qo, co):
    qo[...] = jnp.concatenate([qa[...].T, qb[...].T], axis=1)
    co[...] = jnp.concatenate([ca[...].T, cb[...].T], axis=1)


_pack_call = pl.pallas_call(
    _pack_body,
    grid=(OFF_V // BR2,),
    in_specs=[
        pl.BlockSpec((D, BR2), lambda i: (0, i)),
        pl.BlockSpec((D, BR2), lambda i: (0, jnp.minimum(i + OFF_V // BR2, NBLK))),
        pl.BlockSpec((D, BR2), lambda i: (0, i)),
        pl.BlockSpec((D, BR2), lambda i: (0, jnp.minimum(i + OFF_V // BR2, NBLK))),
    ],
    out_specs=[
        pl.BlockSpec((BR2, 2 * D), lambda i: (i, 0)),
        pl.BlockSpec((BR2, 2 * D), lambda i: (i, 0)),
    ],
    out_shape=[
        jax.ShapeDtypeStruct((OFF_V, 2 * D), jnp.float32),
        jax.ShapeDtypeStruct((OFF_V, 2 * D), jnp.float32),
    ],
)


@functools.partial(
    pl.kernel,
    out_type=(
        jax.ShapeDtypeStruct((B // 2, 2 * D), jnp.float32),
        jax.ShapeDtypeStruct((B // 2, 2 * D), jnp.float32),
    ),
    mesh=_mesh,
    scratch_types=[
        pltpu.VMEM((8, IG), jnp.int32),
        pltpu.VMEM((8, IG), jnp.int32),
        pltpu.VMEM((JB,), jnp.int32),
        pltpu.VMEM((JB,), jnp.int32),
        pltpu.VMEM((JB, 2 * D), jnp.float32),
        pltpu.VMEM((JB, 2 * D), jnp.float32),
        pltpu.VMEM((PAIRS_W, 2 * D), jnp.float32),
        pltpu.SemaphoreType.DMA,
        pltpu.SemaphoreType.DMA,
        pltpu.SemaphoreType.DMA,
        pltpu.SemaphoreType.DMA,
    ],
    compiler_params=pltpu.CompilerParams(
        use_tc_tiling_on_sc=True, needs_layout_passes=False),
)
def _pool_kernel(qidx_hbm, cidx_hbm, tq_hbm, tc_hbm, qout_hbm, cout_hbm,
                 ist0, ist1, sh0, sh1, rows0, rows1, acc_v,
                 si0, si1, sr0, sr1):
    wid = lax.axis_index("s") * NC + lax.axis_index("c")
    prow = pl.multiple_of((wid & ~jnp.int32(1)) * 4, 8)
    srb = (wid & 1) * 4
    ists = (ist0, ist1)
    shs = (sh0, sh1)
    rows = (rows0, rows1)
    sis = (si0, si1)
    srs = (sr0, sr1)
    iota16 = lax.iota(jnp.int32, 16)
    last = jnp.int32(L - 1)

    def plane_copy(idx_hbm, l, pb):
        return pltpu.make_async_copy(
            idx_hbm.at[jnp.minimum(l, last), pl.ds(prow, 8), :], ists[pb], sis[pb])

    def gather_copies(tab_hbm, jb):
        return [
            pltpu.make_async_copy(
                tab_hbm.at[shs[jb].at[pl.ds(g * IG, IG)]],
                rows[jb].at[pl.ds(g * IG, IG), :],
                srs[jb],
            )
            for g in range(JB // IG)
        ]

    def shift(h, pb, jb):
        for k in range(16):
            v = ists[pb][srb + h * 2 + k // 8, pl.ds((k % 8) * 16, 16)]
            shs[jb][pl.ds(k * 16, 16)] = jnp.where(v >= OFF_V, v - OFF_V, v)

    def fire(tab_hbm, jb):
        for cp in gather_copies(tab_hbm, jb):
            cp.start()

    def drain(tab_hbm, jb):
        for cp in gather_copies(tab_hbm, jb):
            cp.wait()

    def accumulate(h, pb, jb):
        rv = rows[jb]
        pv = ists[pb]

        def abody(i2, _):
            for u in range(2):
                ri = i2 * 2 + u
                srow = srb + h * 2 + (ri >> 7)
                lane = ri & 127
                raw = plsc.load_gather(
                    pv, [jnp.full((16,), srow, jnp.int32),
                         jnp.full((16,), lane, jnp.int32)])
                hi_m = raw >= OFF_V
                arow = h * (JB // 2) + i2
                for d in range(4):
                    lo = rv[ri, pl.ds(d * 16, 16)]
                    hi = rv[ri, pl.ds(D + d * 16, 16)]
                    val = jnp.where(hi_m, hi, lo)
                    asl = pl.ds(u * D + d * 16, 16)
                    acc_v[arow, asl] = acc_v[arow, asl] + val
            return 0

        lax.fori_loop(0, JB // 2, abody, 0, unroll=4)

    def zero_acc():
        z = jnp.zeros((16,), jnp.float32)

        def zbody(r, _):
            for c8 in range(8):
                acc_v[r, pl.ds(c8 * 16, 16)] = z
            return 0

        lax.fori_loop(0, PAIRS_W, zbody, 0, unroll=False)

    for idx_hbm, tab_hbm, out_hbm in (
        (qidx_hbm, tq_hbm, qout_hbm),
        (cidx_hbm, tc_hbm, cout_hbm),
    ):
        zero_acc()
        cp = plane_copy(idx_hbm, jnp.int32(0), 0)
        cp.start()
        cp.wait()
        plane_copy(idx_hbm, jnp.int32(1), 1).start()
        shift(0, 0, 0)
        fire(tab_hbm, 0)

        def body(l2, _, idx_hbm=idx_hbm, tab_hbm=tab_hbm):
            sl = l2 * 2
            shift(1, 0, 1)
            fire(tab_hbm, 1)
            drain(tab_hbm, 0)
            accumulate(0, 0, 0)
            plane_copy(idx_hbm, sl + 1, 1).wait()
            shift(0, 1, 0)
            fire(tab_hbm, 0)
            drain(tab_hbm, 1)
            accumulate(1, 0, 1)
            plane_copy(idx_hbm, sl + 2, 0).start()
            shift(1, 1, 1)
            fire(tab_hbm, 1)
            drain(tab_hbm, 0)
            accumulate(0, 1, 0)
            plane_copy(idx_hbm, sl + 2, 0).wait()
            shift(0, 0, 0)
            fire(tab_hbm, 0)
            drain(tab_hbm, 1)
            accumulate(1, 1, 1)
            plane_copy(idx_hbm, sl + 3, 1).start()
            return 0

        lax.fori_loop(0, L // 2, body, 0, unroll=False)
        drain(tab_hbm, 0)
        plane_copy(idx_hbm, last, 1).wait()
        pltpu.sync_copy(
            acc_v,
            out_hbm.at[pl.ds(pl.multiple_of(wid * PAIRS_W, 8), PAIRS_W), :])


BMP = 2048


def _mlp_body(qx, cx, qw1, qb1, qw2, qb2, cw1, cb1, cw2, cb2, qo, co):
    def tower(xp, w1, b1, w2, b2, out_ref):
        xe = xp[:, :D]
        xo = xp[:, D:]
        x = jnp.concatenate([xe, xo], axis=0)
        h = jnp.maximum(
            jnp.dot(x, w1[...], preferred_element_type=jnp.float32) + b1[...], 0.0)
        y = jnp.maximum(
            jnp.dot(h, w2[...], preferred_element_type=jnp.float32) + b2[...], 0.0)
        ye = y[:BMP]
        yo = y[BMP:]
        out_ref[...] = jnp.stack([ye, yo], axis=1).reshape(2 * BMP, H2)

    tower(qx[...], qw1, qb1, qw2, qb2, qo)
    tower(cx[...], cw1, cb1, cw2, cb2, co)


_mlp_call = pl.pallas_call(
    _mlp_body,
    grid=(B // (2 * BMP),),
    in_specs=[
        pl.BlockSpec((BMP, 2 * D), lambda i: (i, 0)),
        pl.BlockSpec((BMP, 2 * D), lambda i: (i, 0)),
        pl.BlockSpec((D, H1), lambda i: (0, 0)),
        pl.BlockSpec((1, H1), lambda i: (0, 0)),
        pl.BlockSpec((H1, H2), lambda i: (0, 0)),
        pl.BlockSpec((1, H2), lambda i: (0, 0)),
        pl.BlockSpec((D, H1), lambda i: (0, 0)),
        pl.BlockSpec((1, H1), lambda i: (0, 0)),
        pl.BlockSpec((H1, H2), lambda i: (0, 0)),
        pl.BlockSpec((1, H2), lambda i: (0, 0)),
    ],
    out_specs=[
        pl.BlockSpec((2 * BMP, H2), lambda i: (i, 0)),
        pl.BlockSpec((2 * BMP, H2), lambda i: (i, 0)),
    ],
    out_shape=[
        jax.ShapeDtypeStruct((B, H2), jnp.float32),
        jax.ShapeDtypeStruct((B, H2), jnp.float32),
    ],
)


def kernel(query_indices, candidate_indices, table_query, table_candidate,
           q_w1, q_b1, q_w2, q_b2, c_w1, c_b1, c_w2, c_b2):
    qidx, cidx = _fmt_call(query_indices.T.astype(jnp.int32),
                           candidate_indices.T.astype(jnp.int32))
    tqT = table_query.T
    tcT = table_candidate.T
    tq2, tc2 = _pack_call(tqT, tqT, tcT, tcT)
    q_pooled, c_pooled = _pool_kernel(qidx, cidx, tq2, tc2)
    qe, ce = _mlp_call(
        q_pooled, c_pooled,
        q_w1, q_b1.reshape(1, H1), q_w2, q_b2.reshape(1, H2),
        c_w1, c_b1.reshape(1, H1), c_w2, c_b2.reshape(1, H2),
    )
    return (qe, ce)
